# baseline (speedup 1.0000x reference)
import math

import jax
import jax.numpy as jnp
from jax.experimental import pallas as pl
from jax.experimental.pallas import tpu as pltpu

BSZ = 2
SEQ = 2048
T = BSZ * SEQ
HID = 2048
NH = 16
NKV = 8
G = NH // NKV
HD = 128
BLK = 64
KB = SEQ // BLK
GH = 128
EPS = 1e-6

TT = 256
QB = 256
CK = 256
NQB = SEQ // QB
BPT = TT // BLK
BF = jnp.bfloat16
F32 = jnp.float32


def _rot(x):
    h = HD // 2
    return jnp.concatenate([-x[..., h:], x[..., :h]], axis=-1)


def _proj_kernel(h_ref, cos_ref, sin_ref, cg_ref, sg_ref, bc_ref, bs_ref,
                 wq_ref, wkv_ref, qw_ref, kw_ref, gwq_ref, gwk_ref,
                 qr_ref, kr_ref, v_ref, qg_ref, kg_ref):
    t = pl.program_id(1)
    lt = jax.lax.rem(t, SEQ // TT)
    ht = h_ref[pl.ds(t * TT, TT), :]
    cos = cos_ref[pl.ds(t * TT, TT), :]
    sin = sin_ref[pl.ds(t * TT, TT), :]

    q = jnp.dot(ht, wq_ref[0], preferred_element_type=F32)
    q3 = q.reshape(TT, G, HD)
    var = jnp.mean(q3 * q3, axis=-1, keepdims=True)
    qn = q3 * jax.lax.rsqrt(var + EPS) * qw_ref[0][None, None, :]
    qr = qn * cos[:, None, :] + _rot(qn) * sin[:, None, :]
    qr_ref[0, 0] = qr.reshape(TT, G * HD).astype(BF)

    cg = cg_ref[pl.ds(t * TT, TT), :]
    sg = sg_ref[pl.ds(t * TT, TT), :]
    qg = qn.mean(axis=1)
    qg = qg * cg + _rot(qg) * sg
    qg_ref[0, 0] = jnp.dot(qg.astype(BF), gwq_ref[...],
                           preferred_element_type=F32).astype(BF)

    kv = jnp.dot(ht, wkv_ref[0], preferred_element_type=F32)
    k = kv[:, :HD]
    kvar = jnp.mean(k * k, axis=-1, keepdims=True)
    kn = k * jax.lax.rsqrt(kvar + EPS) * kw_ref[0][None, :]
    kr_ref[0, 0] = (kn * cos + _rot(kn) * sin).astype(BF)
    v_ref[0, 0] = kv[:, HD:].astype(BF)

    kb3 = kn.reshape(BPT, BLK, HD)
    kmax = kb3.max(axis=1)
    kavg = kb3.mean(axis=1)
    bc = bc_ref[pl.ds(lt * BPT, BPT), :]
    bs = bs_ref[pl.ds(lt * BPT, BPT), :]
    kmax = kmax * bc + _rot(kmax) * bs
    kavg = kavg * bc + _rot(kavg) * bs
    kcat = jnp.concatenate([kmax, kavg], axis=-1)
    kg_ref[0, 0, 0] = jnp.dot(kcat.astype(BF), gwk_ref[...],
                              preferred_element_type=F32).astype(BF)


def _flash_kernel(q_ref, k_ref, v_ref, qg_ref, kg_ref, o_ref, kl_ref):
    qb = pl.program_id(1)
    scale = 1.0 / math.sqrt(HD)
    rows = G * QB

    qblk = q_ref[0, 0, :, :]
    qs = jnp.concatenate([qblk[:, :HD], qblk[:, HD:]], axis=0)

    ind = (jax.lax.broadcasted_iota(jnp.int32, (CK, CK // BLK), 1)
           == jax.lax.broadcasted_iota(jnp.int32, (CK, CK // BLK), 0)
           // BLK).astype(BF)

    def chunk(j, m, l, acc, psum, masked):
        kc = k_ref[0, 0, pl.ds(j * CK, CK), :]
        vc = v_ref[0, 0, pl.ds(j * CK, CK), :]
        s = jax.lax.dot_general(qs, kc, (((1,), (1,)), ((), ())),
                                preferred_element_type=F32) * scale
        if masked:
            rq = jax.lax.broadcasted_iota(jnp.int32, (rows, CK), 0) % QB
            ck = jax.lax.broadcasted_iota(jnp.int32, (rows, CK), 1)
            s = jnp.where(ck <= rq, s, -1e30)
        m_new = jnp.maximum(m, s.max(axis=-1, keepdims=True))
        corr = jnp.exp(m - m_new)
        p = jnp.exp(s - m_new)
        pb = p.astype(BF)
        l = l * corr + p.sum(axis=-1, keepdims=True)
        acc = acc * corr + jnp.dot(pb, vc, preferred_element_type=F32)
        ps = jnp.dot(pb, ind, preferred_element_type=F32)
        colid = jax.lax.broadcasted_iota(jnp.int32, (rows, KB), 1)
        upd = jnp.zeros((rows, KB), dtype=F32)
        for c in range(CK // BLK):
            upd = upd + jnp.where(colid == j * (CK // BLK) + c,
                                  ps[:, c][:, None], 0.0)
        psum = psum * corr + upd
        return m_new, l, acc, psum

    m0 = jnp.full((rows, 1), -1e30, dtype=F32)
    l0 = jnp.zeros((rows, 1), dtype=F32)
    a0 = jnp.zeros((rows, HD), dtype=F32)
    p0 = jnp.zeros((rows, KB), dtype=F32)

    def body(j, carry):
        return chunk(j, *carry, masked=False)

    m, l, acc, psum = jax.lax.fori_loop(0, qb, body, (m0, l0, a0, p0))
    m, l, acc, psum = chunk(qb, m, l, acc, psum, masked=True)

    attn = acc / l
    o_ref[0, 0, :, :] = jnp.concatenate(
        [attn[:QB], attn[QB:]], axis=1).astype(BF)

    pn = psum / l
    m1d = jnp.maximum(pn[:QB], pn[QB:])
    gt = m1d / (m1d.sum(axis=-1, keepdims=True) + 1e-9)

    qg = qg_ref[0, 0, :, :]
    kg = kg_ref[0, 0, :, :]
    logits = jax.lax.dot_general(qg, kg, (((1,), (1,)), ((), ())),
                                 preferred_element_type=F32)
    logits = logits * (1.0 / math.sqrt(GH))
    rowq = jax.lax.broadcasted_iota(jnp.int32, (QB, KB), 0) + qb * QB
    colb = jax.lax.broadcasted_iota(jnp.int32, (QB, KB), 1) * BLK
    x = jnp.where(colb <= rowq, logits, -1e30)
    xm = x.max(axis=-1, keepdims=True)
    pm = x - xm - jnp.log(jnp.exp(x - xm).sum(axis=-1, keepdims=True))

    gt_safe = jnp.where(gt > 0, gt, 1.0)
    kl = jnp.where(gt > 0, gt * (jnp.log(gt_safe) - pm), 0.0)
    kl_ref[0, 0, :] = jnp.full((GH,), kl.sum(), dtype=F32)


def _out_kernel(x_ref, wo_ref, o_ref):
    n = pl.program_id(1)
    part = jnp.dot(x_ref[0, 0], wo_ref[pl.ds(n * G * HD, G * HD), :],
                   preferred_element_type=F32)

    @pl.when(n == 0)
    def _():
        o_ref[...] = part

    @pl.when(n != 0)
    def _():
        o_ref[...] += part


def kernel(hidden_states, cos, sin, cos_gate, sin_gate, block_cos, block_sin,
           block_attention_mask, cu_seqlens, wq, wk, wv, wo,
           q_norm_w, k_norm_w, gate_wq, gate_wk):
    nt = T // TT
    ntb = SEQ // TT

    wq3 = wq.reshape(HID, NKV, G * HD).transpose(1, 0, 2).astype(BF)
    wkv3 = jnp.concatenate(
        [wk.reshape(HID, NKV, 1, HD), wv.reshape(HID, NKV, 1, HD)],
        axis=2).reshape(HID, NKV, 2 * HD).transpose(1, 0, 2).astype(BF)

    qr4, kr4, v4, qg4, kg5 = pl.pallas_call(
        _proj_kernel,
        grid=(NKV, nt),
        in_specs=[
            pl.BlockSpec((T, HID), lambda n, t: (0, 0)),
            pl.BlockSpec((T, HD), lambda n, t: (0, 0)),
            pl.BlockSpec((T, HD), lambda n, t: (0, 0)),
            pl.BlockSpec((T, HD), lambda n, t: (0, 0)),
            pl.BlockSpec((T, HD), lambda n, t: (0, 0)),
            pl.BlockSpec((KB, HD), lambda n, t: (0, 0)),
            pl.BlockSpec((KB, HD), lambda n, t: (0, 0)),
            pl.BlockSpec((1, HID, G * HD), lambda n, t: (n, 0, 0)),
            pl.BlockSpec((1, HID, 2 * HD), lambda n, t: (n, 0, 0)),
            pl.BlockSpec((1, HD), lambda n, t: (0, 0)),
            pl.BlockSpec((1, HD), lambda n, t: (0, 0)),
            pl.BlockSpec((HD, GH), lambda n, t: (0, 0)),
            pl.BlockSpec((2 * HD, GH), lambda n, t: (0, 0)),
        ],
        out_specs=[
            pl.BlockSpec((1, 1, TT, G * HD),
                         lambda n, t: (t // ntb, n, t % ntb, 0)),
            pl.BlockSpec((1, 1, TT, HD),
                         lambda n, t: (t // ntb, n, t % ntb, 0)),
            pl.BlockSpec((1, 1, TT, HD),
                         lambda n, t: (t // ntb, n, t % ntb, 0)),
            pl.BlockSpec((1, 1, TT, GH),
                         lambda n, t: (t // ntb, n, t % ntb, 0)),
            pl.BlockSpec((1, 1, 1, BPT, GH),
                         lambda n, t: (t // ntb, n, t % ntb, 0, 0)),
        ],
        out_shape=[
            jax.ShapeDtypeStruct((BSZ, NKV, SEQ, G * HD), BF),
            jax.ShapeDtypeStruct((BSZ, NKV, SEQ, HD), BF),
            jax.ShapeDtypeStruct((BSZ, NKV, SEQ, HD), BF),
            jax.ShapeDtypeStruct((BSZ, NKV, SEQ, GH), BF),
            jax.ShapeDtypeStruct((BSZ, NKV, ntb, BPT, GH), BF),
        ],
        compiler_params=pltpu.CompilerParams(
            dimension_semantics=("parallel", "arbitrary")),
    )(hidden_states.astype(BF), cos, sin, cos_gate, sin_gate,
      block_cos, block_sin, wq3, wkv3,
      q_norm_w.reshape(1, HD), k_norm_w.reshape(1, HD),
      gate_wq.astype(BF), gate_wk.astype(BF))

    kg4 = kg5.reshape(BSZ, NKV, KB, GH)

    attn, klp = pl.pallas_call(
        _flash_kernel,
        grid=(BSZ * NKV, NQB),
        in_specs=[
            pl.BlockSpec((1, 1, QB, G * HD),
                         lambda bn, qb: (bn // NKV, bn % NKV, qb, 0)),
            pl.BlockSpec((1, 1, SEQ, HD),
                         lambda bn, qb: (bn // NKV, bn % NKV, 0, 0)),
            pl.BlockSpec((1, 1, SEQ, HD),
                         lambda bn, qb: (bn // NKV, bn % NKV, 0, 0)),
            pl.BlockSpec((1, 1, QB, GH),
                         lambda bn, qb: (bn // NKV, bn % NKV, qb, 0)),
            pl.BlockSpec((1, 1, KB, GH),
                         lambda bn, qb: (bn // NKV, bn % NKV, 0, 0)),
        ],
        out_specs=[
            pl.BlockSpec((1, 1, QB, G * HD),
                         lambda bn, qb: (bn // NKV, bn % NKV, qb, 0)),
            pl.BlockSpec((1, 1, GH), lambda bn, qb: (bn * NQB + qb, 0, 0)),
        ],
        out_shape=[
            jax.ShapeDtypeStruct((BSZ, NKV, SEQ, G * HD), BF),
            jax.ShapeDtypeStruct((BSZ * NKV * NQB, 1, GH), F32),
        ],
        compiler_params=pltpu.CompilerParams(
            dimension_semantics=("parallel", "arbitrary")),
    )(qr4, kr4, v4, qg4, kg4)

    attn_output = pl.pallas_call(
        _out_kernel,
        grid=(nt, NKV),
        in_specs=[
            pl.BlockSpec((1, 1, TT, G * HD),
                         lambda t, n: (t // ntb, n, t % ntb, 0)),
            pl.BlockSpec((NH * HD, HID), lambda t, n: (0, 0)),
        ],
        out_specs=pl.BlockSpec((TT, HID), lambda t, n: (t, 0)),
        out_shape=jax.ShapeDtypeStruct((T, HID), F32),
        compiler_params=pltpu.CompilerParams(
            dimension_semantics=("parallel", "arbitrary")),
    )(attn, wo.astype(BF))

    gate_loss = klp[:, 0, 0].sum() * (1.0 / (NKV * SEQ * KB * BSZ))
    return attn_output, gate_loss


# speedup vs baseline: 1.2201x; 1.2201x over previous
import math

import jax
import jax.numpy as jnp
from jax.experimental import pallas as pl
from jax.experimental.pallas import tpu as pltpu

BSZ = 2
SEQ = 2048
T = BSZ * SEQ
HID = 2048
NH = 16
NKV = 8
G = NH // NKV
HD = 128
BLK = 64
KB = SEQ // BLK
GH = 128
EPS = 1e-6

TT = 256
QB = 256
CK = 256
NQB = SEQ // QB
BPT = TT // BLK
BF = jnp.bfloat16
F32 = jnp.float32


def _rot(x):
    h = HD // 2
    return jnp.concatenate([-x[..., h:], x[..., :h]], axis=-1)


def _proj_kernel(h_ref, cos_ref, sin_ref, cg_ref, sg_ref, bc_ref, bs_ref,
                 wq_ref, wk_ref, wv_ref, qw_ref, kw_ref, gwq_ref, gwk_ref,
                 qr_ref, kr_ref, v_ref, qg_ref, kg_ref):
    h = h_ref[...]
    cos = cos_ref[...]
    sin = sin_ref[...]

    q = jnp.dot(h, wq_ref[...], preferred_element_type=F32)
    q3 = q.reshape(TT, NH, HD)
    var = jnp.mean(q3 * q3, axis=-1, keepdims=True)
    qn = q3 * jax.lax.rsqrt(var + EPS) * qw_ref[0][None, None, :]
    qr = qn * cos[:, None, :] + _rot(qn) * sin[:, None, :]
    qr_ref[...] = qr.reshape(TT, NH * HD).astype(BF)

    qg = qn.reshape(TT, NKV, G, HD).mean(axis=2)
    qg = qg * cg_ref[...][:, None, :] + _rot(qg) * sg_ref[...][:, None, :]
    qgp = jnp.dot(qg.reshape(TT * NKV, HD).astype(BF), gwq_ref[...],
                  preferred_element_type=F32)
    qg_ref[...] = qgp.reshape(TT, NKV * GH).astype(BF)

    k = jnp.dot(h, wk_ref[...], preferred_element_type=F32)
    k3 = k.reshape(TT, NKV, HD)
    kvar = jnp.mean(k3 * k3, axis=-1, keepdims=True)
    kn = k3 * jax.lax.rsqrt(kvar + EPS) * kw_ref[0][None, None, :]
    kr = kn * cos[:, None, :] + _rot(kn) * sin[:, None, :]
    kr_ref[...] = kr.reshape(TT, NKV * HD).astype(BF)

    kb4 = kn.reshape(BPT, BLK, NKV, HD)
    kmax = kb4.max(axis=1)
    kavg = kb4.mean(axis=1)
    bc = bc_ref[:, 0, :]
    bs = bs_ref[:, 0, :]
    kmax = kmax * bc[:, None, :] + _rot(kmax) * bs[:, None, :]
    kavg = kavg * bc[:, None, :] + _rot(kavg) * bs[:, None, :]
    kcat = jnp.concatenate([kmax, kavg], axis=-1).reshape(BPT * NKV, 2 * HD)
    kgp = jnp.dot(kcat.astype(BF), gwk_ref[...], preferred_element_type=F32)
    kg_ref[:, 0, :] = kgp.reshape(BPT, NKV * GH).astype(BF)

    v_ref[...] = jnp.dot(h, wv_ref[...],
                         preferred_element_type=F32).astype(BF)


def _flash_kernel(q_ref, k_ref, v_ref, qg_ref, kg_ref, o_ref, kl_ref):
    qb = pl.program_id(1)
    scale = 1.0 / math.sqrt(HD)
    rows = G * QB

    qblk = q_ref[0, 0, :, :]
    qs = jnp.concatenate([qblk[:, :HD], qblk[:, HD:]], axis=0)

    ind = (jax.lax.broadcasted_iota(jnp.int32, (CK, CK // BLK), 1)
           == jax.lax.broadcasted_iota(jnp.int32, (CK, CK // BLK), 0)
           // BLK).astype(BF)

    def chunk(j, m, l, acc, psum, masked):
        kc = k_ref[0, 0, pl.ds(j * CK, CK), :]
        vc = v_ref[0, 0, pl.ds(j * CK, CK), :]
        s = jax.lax.dot_general(qs, kc, (((1,), (1,)), ((), ())),
                                preferred_element_type=F32) * scale
        if masked:
            rq = jax.lax.broadcasted_iota(jnp.int32, (rows, CK), 0) % QB
            ck = jax.lax.broadcasted_iota(jnp.int32, (rows, CK), 1)
            s = jnp.where(ck <= rq, s, -1e30)
        m_new = jnp.maximum(m, s.max(axis=-1, keepdims=True))
        corr = jnp.exp(m - m_new)
        p = jnp.exp(s - m_new)
        pb = p.astype(BF)
        l = l * corr + p.sum(axis=-1, keepdims=True)
        acc = acc * corr + jnp.dot(pb, vc, preferred_element_type=F32)
        ps = jnp.dot(pb, ind, preferred_element_type=F32)
        colid = jax.lax.broadcasted_iota(jnp.int32, (rows, KB), 1)
        upd = jnp.zeros((rows, KB), dtype=F32)
        for c in range(CK // BLK):
            upd = upd + jnp.where(colid == j * (CK // BLK) + c,
                                  ps[:, c][:, None], 0.0)
        psum = psum * corr + upd
        return m_new, l, acc, psum

    m0 = jnp.full((rows, 1), -1e30, dtype=F32)
    l0 = jnp.zeros((rows, 1), dtype=F32)
    a0 = jnp.zeros((rows, HD), dtype=F32)
    p0 = jnp.zeros((rows, KB), dtype=F32)

    def body(j, carry):
        return chunk(j, *carry, masked=False)

    m, l, acc, psum = jax.lax.fori_loop(0, qb, body, (m0, l0, a0, p0))
    m, l, acc, psum = chunk(qb, m, l, acc, psum, masked=True)

    attn = acc / l
    o_ref[0, 0, :, :] = jnp.concatenate(
        [attn[:QB], attn[QB:]], axis=1).astype(BF)

    pn = psum / l
    m1d = jnp.maximum(pn[:QB], pn[QB:])
    gt = m1d / (m1d.sum(axis=-1, keepdims=True) + 1e-9)

    qg = qg_ref[0, 0, :, :]
    kg = kg_ref[0, 0, :, :]
    logits = jax.lax.dot_general(qg, kg, (((1,), (1,)), ((), ())),
                                 preferred_element_type=F32)
    logits = logits * (1.0 / math.sqrt(GH))
    rowq = jax.lax.broadcasted_iota(jnp.int32, (QB, KB), 0) + qb * QB
    colb = jax.lax.broadcasted_iota(jnp.int32, (QB, KB), 1) * BLK
    x = jnp.where(colb <= rowq, logits, -1e30)
    xm = x.max(axis=-1, keepdims=True)
    pm = x - xm - jnp.log(jnp.exp(x - xm).sum(axis=-1, keepdims=True))

    gt_safe = jnp.where(gt > 0, gt, 1.0)
    kl = jnp.where(gt > 0, gt * (jnp.log(gt_safe) - pm), 0.0)
    kl_ref[0, 0, :] = jnp.full((GH,), kl.sum(), dtype=F32)


def _out_kernel(x_ref, wo_ref, o_ref):
    o_ref[...] = jnp.dot(x_ref[...], wo_ref[...],
                         preferred_element_type=F32)


def kernel(hidden_states, cos, sin, cos_gate, sin_gate, block_cos, block_sin,
           block_attention_mask, cu_seqlens, wq, wk, wv, wo,
           q_norm_w, k_norm_w, gate_wq, gate_wk):
    nt = T // TT
    ntb = SEQ // TT

    qr, kr, v, qg, kg = pl.pallas_call(
        _proj_kernel,
        grid=(nt,),
        in_specs=[
            pl.BlockSpec((TT, HID), lambda t: (t, 0)),
            pl.BlockSpec((TT, HD), lambda t: (t, 0)),
            pl.BlockSpec((TT, HD), lambda t: (t, 0)),
            pl.BlockSpec((TT, HD), lambda t: (t, 0)),
            pl.BlockSpec((TT, HD), lambda t: (t, 0)),
            pl.BlockSpec((BPT, 1, HD), lambda t: (t % (SEQ // TT), 0, 0)),
            pl.BlockSpec((BPT, 1, HD), lambda t: (t % (SEQ // TT), 0, 0)),
            pl.BlockSpec((HID, NH * HD), lambda t: (0, 0)),
            pl.BlockSpec((HID, NKV * HD), lambda t: (0, 0)),
            pl.BlockSpec((HID, NKV * HD), lambda t: (0, 0)),
            pl.BlockSpec((1, HD), lambda t: (0, 0)),
            pl.BlockSpec((1, HD), lambda t: (0, 0)),
            pl.BlockSpec((HD, GH), lambda t: (0, 0)),
            pl.BlockSpec((2 * HD, GH), lambda t: (0, 0)),
        ],
        out_specs=[
            pl.BlockSpec((TT, NH * HD), lambda t: (t, 0)),
            pl.BlockSpec((TT, NKV * HD), lambda t: (t, 0)),
            pl.BlockSpec((TT, NKV * HD), lambda t: (t, 0)),
            pl.BlockSpec((TT, NKV * GH), lambda t: (t, 0)),
            pl.BlockSpec((BPT, 1, NKV * GH), lambda t: (t, 0, 0)),
        ],
        out_shape=[
            jax.ShapeDtypeStruct((T, NH * HD), BF),
            jax.ShapeDtypeStruct((T, NKV * HD), BF),
            jax.ShapeDtypeStruct((T, NKV * HD), BF),
            jax.ShapeDtypeStruct((T, NKV * GH), BF),
            jax.ShapeDtypeStruct((BSZ * KB, 1, NKV * GH), BF),
        ],
        compiler_params=pltpu.CompilerParams(
            dimension_semantics=("parallel",)),
    )(hidden_states.astype(BF), cos, sin, cos_gate, sin_gate,
      block_cos.reshape(KB, 1, HD), block_sin.reshape(KB, 1, HD),
      wq.astype(BF), wk.astype(BF), wv.astype(BF),
      q_norm_w.reshape(1, HD), k_norm_w.reshape(1, HD),
      gate_wq.astype(BF), gate_wk.astype(BF))

    qr4 = qr.reshape(BSZ, SEQ, NKV, G * HD).transpose(0, 2, 1, 3)
    kr4 = kr.reshape(BSZ, SEQ, NKV, HD).transpose(0, 2, 1, 3)
    v4 = v.reshape(BSZ, SEQ, NKV, HD).transpose(0, 2, 1, 3)
    qg4 = qg.reshape(BSZ, SEQ, NKV, GH).transpose(0, 2, 1, 3)
    kg4 = kg.reshape(BSZ, KB, NKV, GH).transpose(0, 2, 1, 3)

    attn, klp = pl.pallas_call(
        _flash_kernel,
        grid=(BSZ * NKV, NQB),
        in_specs=[
            pl.BlockSpec((1, 1, QB, G * HD),
                         lambda bn, qb: (bn // NKV, bn % NKV, qb, 0)),
            pl.BlockSpec((1, 1, SEQ, HD),
                         lambda bn, qb: (bn // NKV, bn % NKV, 0, 0)),
            pl.BlockSpec((1, 1, SEQ, HD),
                         lambda bn, qb: (bn // NKV, bn % NKV, 0, 0)),
            pl.BlockSpec((1, 1, QB, GH),
                         lambda bn, qb: (bn // NKV, bn % NKV, qb, 0)),
            pl.BlockSpec((1, 1, KB, GH),
                         lambda bn, qb: (bn // NKV, bn % NKV, 0, 0)),
        ],
        out_specs=[
            pl.BlockSpec((1, 1, QB, G * HD),
                         lambda bn, qb: (bn // NKV, bn % NKV, qb, 0)),
            pl.BlockSpec((1, 1, GH), lambda bn, qb: (bn * NQB + qb, 0, 0)),
        ],
        out_shape=[
            jax.ShapeDtypeStruct((BSZ, NKV, SEQ, G * HD), BF),
            jax.ShapeDtypeStruct((BSZ * NKV * NQB, 1, GH), F32),
        ],
        compiler_params=pltpu.CompilerParams(
            dimension_semantics=("parallel", "arbitrary")),
    )(qr4, kr4, v4, qg4, kg4)

    attn_output = pl.pallas_call(
        _out_kernel,
        grid=(nt,),
        in_specs=[
            pl.BlockSpec((TT, NH * HD), lambda t: (t, 0)),
            pl.BlockSpec((NH * HD, HID), lambda t: (0, 0)),
        ],
        out_specs=pl.BlockSpec((TT, HID), lambda t: (t, 0)),
        out_shape=jax.ShapeDtypeStruct((T, HID), F32),
        compiler_params=pltpu.CompilerParams(
            dimension_semantics=("parallel",)),
    )(attn.transpose(0, 2, 1, 3).reshape(T, NH * HD), wo.astype(BF))

    gate_loss = klp[:, 0, 0].sum() * (1.0 / (NKV * SEQ * KB * BSZ))
    return attn_output, gate_loss


# speedup vs baseline: 1.3306x; 1.0906x over previous
import math

import jax
import jax.numpy as jnp
from jax.experimental import pallas as pl
from jax.experimental.pallas import tpu as pltpu

BSZ = 2
SEQ = 2048
T = BSZ * SEQ
HID = 2048
NH = 16
NKV = 8
G = NH // NKV
HD = 128
BLK = 64
KB = SEQ // BLK
GH = 128
EPS = 1e-6

TT = 256
QB = 512
CK = 512
NQB = SEQ // QB
BPT = TT // BLK
BF = jnp.bfloat16
F32 = jnp.float32


def _rot(x):
    h = HD // 2
    return jnp.concatenate([-x[..., h:], x[..., :h]], axis=-1)


def _proj_kernel(h_ref, cos_ref, sin_ref, cg_ref, sg_ref, bc_ref, bs_ref,
                 wq_ref, wk_ref, wv_ref, qw_ref, kw_ref, gwq_ref, gwk_ref,
                 qr_ref, kr_ref, v_ref, qg_ref, kg_ref):
    h = h_ref[...]
    cos = cos_ref[...]
    sin = sin_ref[...]

    q = jnp.dot(h, wq_ref[...], preferred_element_type=F32)
    q3 = q.reshape(TT, NH, HD)
    var = jnp.mean(q3 * q3, axis=-1, keepdims=True)
    qn = q3 * jax.lax.rsqrt(var + EPS) * qw_ref[0][None, None, :]
    qr = qn * cos[:, None, :] + _rot(qn) * sin[:, None, :]
    qr_ref[...] = qr.reshape(TT, NH * HD).astype(BF)

    qg = qn.reshape(TT, NKV, G, HD).mean(axis=2)
    qg = qg * cg_ref[...][:, None, :] + _rot(qg) * sg_ref[...][:, None, :]
    qgp = jnp.dot(qg.reshape(TT * NKV, HD).astype(BF), gwq_ref[...],
                  preferred_element_type=F32)
    qg_ref[...] = qgp.reshape(TT, NKV * GH).astype(BF)

    k = jnp.dot(h, wk_ref[...], preferred_element_type=F32)
    k3 = k.reshape(TT, NKV, HD)
    kvar = jnp.mean(k3 * k3, axis=-1, keepdims=True)
    kn = k3 * jax.lax.rsqrt(kvar + EPS) * kw_ref[0][None, None, :]
    kr = kn * cos[:, None, :] + _rot(kn) * sin[:, None, :]
    kr_ref[...] = kr.reshape(TT, NKV * HD).astype(BF)

    kb4 = kn.reshape(BPT, BLK, NKV, HD)
    kmax = kb4.max(axis=1)
    kavg = kb4.mean(axis=1)
    bc = bc_ref[:, 0, :]
    bs = bs_ref[:, 0, :]
    kmax = kmax * bc[:, None, :] + _rot(kmax) * bs[:, None, :]
    kavg = kavg * bc[:, None, :] + _rot(kavg) * bs[:, None, :]
    kcat = jnp.concatenate([kmax, kavg], axis=-1).reshape(BPT * NKV, 2 * HD)
    kgp = jnp.dot(kcat.astype(BF), gwk_ref[...], preferred_element_type=F32)
    kg_ref[:, 0, :] = kgp.reshape(BPT, NKV * GH).astype(BF)

    v_ref[...] = jnp.dot(h, wv_ref[...],
                         preferred_element_type=F32).astype(BF)


def _flash_kernel(q_ref, k_ref, v_ref, qg_ref, kg_ref, o_ref, kl_ref):
    qb = pl.program_id(1)
    scale = 1.0 / math.sqrt(HD)
    rows = G * QB

    qblk = q_ref[0, 0, :, :]
    qs = jnp.concatenate([qblk[:, :HD], qblk[:, HD:]], axis=0)

    ind = (jax.lax.broadcasted_iota(jnp.int32, (CK, CK // BLK), 1)
           == jax.lax.broadcasted_iota(jnp.int32, (CK, CK // BLK), 0)
           // BLK).astype(BF)

    def chunk(j, m, l, acc, psum, masked):
        kc = k_ref[0, 0, pl.ds(j * CK, CK), :]
        vc = v_ref[0, 0, pl.ds(j * CK, CK), :]
        s = jax.lax.dot_general(qs, kc, (((1,), (1,)), ((), ())),
                                preferred_element_type=F32) * scale
        if masked:
            rq = jax.lax.broadcasted_iota(jnp.int32, (rows, CK), 0) % QB
            ck = jax.lax.broadcasted_iota(jnp.int32, (rows, CK), 1)
            s = jnp.where(ck <= rq, s, -1e30)
        m_new = jnp.maximum(m, s.max(axis=-1, keepdims=True))
        corr = jnp.exp(m - m_new)
        p = jnp.exp(s - m_new)
        pb = p.astype(BF)
        l = l * corr + p.sum(axis=-1, keepdims=True)
        acc = acc * corr + jnp.dot(pb, vc, preferred_element_type=F32)
        ps = jnp.dot(pb, ind, preferred_element_type=F32)
        colid = jax.lax.broadcasted_iota(jnp.int32, (rows, KB), 1)
        upd = jnp.zeros((rows, KB), dtype=F32)
        for c in range(CK // BLK):
            upd = upd + jnp.where(colid == j * (CK // BLK) + c,
                                  ps[:, c][:, None], 0.0)
        psum = psum * corr + upd
        return m_new, l, acc, psum

    m0 = jnp.full((rows, 1), -1e30, dtype=F32)
    l0 = jnp.zeros((rows, 1), dtype=F32)
    a0 = jnp.zeros((rows, HD), dtype=F32)
    p0 = jnp.zeros((rows, KB), dtype=F32)

    def body(j, carry):
        return chunk(j, *carry, masked=False)

    m, l, acc, psum = jax.lax.fori_loop(0, qb, body, (m0, l0, a0, p0))
    m, l, acc, psum = chunk(qb, m, l, acc, psum, masked=True)

    attn = acc / l
    o_ref[0, 0, :, :] = jnp.concatenate(
        [attn[:QB], attn[QB:]], axis=1).astype(BF)

    pn = psum / l
    m1d = jnp.maximum(pn[:QB], pn[QB:])
    gt = m1d / (m1d.sum(axis=-1, keepdims=True) + 1e-9)

    qg = qg_ref[0, 0, :, :]
    kg = kg_ref[0, 0, :, :]
    logits = jax.lax.dot_general(qg, kg, (((1,), (1,)), ((), ())),
                                 preferred_element_type=F32)
    logits = logits * (1.0 / math.sqrt(GH))
    rowq = jax.lax.broadcasted_iota(jnp.int32, (QB, KB), 0) + qb * QB
    colb = jax.lax.broadcasted_iota(jnp.int32, (QB, KB), 1) * BLK
    x = jnp.where(colb <= rowq, logits, -1e30)
    xm = x.max(axis=-1, keepdims=True)
    pm = x - xm - jnp.log(jnp.exp(x - xm).sum(axis=-1, keepdims=True))

    gt_safe = jnp.where(gt > 0, gt, 1.0)
    kl = jnp.where(gt > 0, gt * (jnp.log(gt_safe) - pm), 0.0)
    kl_ref[0, 0, :] = jnp.full((GH,), kl.sum(), dtype=F32)


def _out_kernel(x_ref, wo_ref, o_ref):
    o_ref[...] = jnp.dot(x_ref[...], wo_ref[...],
                         preferred_element_type=F32)


def kernel(hidden_states, cos, sin, cos_gate, sin_gate, block_cos, block_sin,
           block_attention_mask, cu_seqlens, wq, wk, wv, wo,
           q_norm_w, k_norm_w, gate_wq, gate_wk):
    nt = T // TT
    ntb = SEQ // TT

    qr, kr, v, qg, kg = pl.pallas_call(
        _proj_kernel,
        grid=(nt,),
        in_specs=[
            pl.BlockSpec((TT, HID), lambda t: (t, 0)),
            pl.BlockSpec((TT, HD), lambda t: (t, 0)),
            pl.BlockSpec((TT, HD), lambda t: (t, 0)),
            pl.BlockSpec((TT, HD), lambda t: (t, 0)),
            pl.BlockSpec((TT, HD), lambda t: (t, 0)),
            pl.BlockSpec((BPT, 1, HD), lambda t: (t % (SEQ // TT), 0, 0)),
            pl.BlockSpec((BPT, 1, HD), lambda t: (t % (SEQ // TT), 0, 0)),
            pl.BlockSpec((HID, NH * HD), lambda t: (0, 0)),
            pl.BlockSpec((HID, NKV * HD), lambda t: (0, 0)),
            pl.BlockSpec((HID, NKV * HD), lambda t: (0, 0)),
            pl.BlockSpec((1, HD), lambda t: (0, 0)),
            pl.BlockSpec((1, HD), lambda t: (0, 0)),
            pl.BlockSpec((HD, GH), lambda t: (0, 0)),
            pl.BlockSpec((2 * HD, GH), lambda t: (0, 0)),
        ],
        out_specs=[
            pl.BlockSpec((TT, NH * HD), lambda t: (t, 0)),
            pl.BlockSpec((TT, NKV * HD), lambda t: (t, 0)),
            pl.BlockSpec((TT, NKV * HD), lambda t: (t, 0)),
            pl.BlockSpec((TT, NKV * GH), lambda t: (t, 0)),
            pl.BlockSpec((BPT, 1, NKV * GH), lambda t: (t, 0, 0)),
        ],
        out_shape=[
            jax.ShapeDtypeStruct((T, NH * HD), BF),
            jax.ShapeDtypeStruct((T, NKV * HD), BF),
            jax.ShapeDtypeStruct((T, NKV * HD), BF),
            jax.ShapeDtypeStruct((T, NKV * GH), BF),
            jax.ShapeDtypeStruct((BSZ * KB, 1, NKV * GH), BF),
        ],
        compiler_params=pltpu.CompilerParams(
            dimension_semantics=("parallel",)),
    )(hidden_states.astype(BF), cos, sin, cos_gate, sin_gate,
      block_cos.reshape(KB, 1, HD), block_sin.reshape(KB, 1, HD),
      wq.astype(BF), wk.astype(BF), wv.astype(BF),
      q_norm_w.reshape(1, HD), k_norm_w.reshape(1, HD),
      gate_wq.astype(BF), gate_wk.astype(BF))

    qr4 = qr.reshape(BSZ, SEQ, NKV, G * HD).transpose(0, 2, 1, 3)
    kr4 = kr.reshape(BSZ, SEQ, NKV, HD).transpose(0, 2, 1, 3)
    v4 = v.reshape(BSZ, SEQ, NKV, HD).transpose(0, 2, 1, 3)
    qg4 = qg.reshape(BSZ, SEQ, NKV, GH).transpose(0, 2, 1, 3)
    kg4 = kg.reshape(BSZ, KB, NKV, GH).transpose(0, 2, 1, 3)

    attn, klp = pl.pallas_call(
        _flash_kernel,
        grid=(BSZ * NKV, NQB),
        in_specs=[
            pl.BlockSpec((1, 1, QB, G * HD),
                         lambda bn, qb: (bn // NKV, bn % NKV, qb, 0)),
            pl.BlockSpec((1, 1, SEQ, HD),
                         lambda bn, qb: (bn // NKV, bn % NKV, 0, 0)),
            pl.BlockSpec((1, 1, SEQ, HD),
                         lambda bn, qb: (bn // NKV, bn % NKV, 0, 0)),
            pl.BlockSpec((1, 1, QB, GH),
                         lambda bn, qb: (bn // NKV, bn % NKV, qb, 0)),
            pl.BlockSpec((1, 1, KB, GH),
                         lambda bn, qb: (bn // NKV, bn % NKV, 0, 0)),
        ],
        out_specs=[
            pl.BlockSpec((1, 1, QB, G * HD),
                         lambda bn, qb: (bn // NKV, bn % NKV, qb, 0)),
            pl.BlockSpec((1, 1, GH), lambda bn, qb: (bn * NQB + qb, 0, 0)),
        ],
        out_shape=[
            jax.ShapeDtypeStruct((BSZ, NKV, SEQ, G * HD), BF),
            jax.ShapeDtypeStruct((BSZ * NKV * NQB, 1, GH), F32),
        ],
        compiler_params=pltpu.CompilerParams(
            dimension_semantics=("parallel", "arbitrary")),
    )(qr4, kr4, v4, qg4, kg4)

    attn_output = pl.pallas_call(
        _out_kernel,
        grid=(nt,),
        in_specs=[
            pl.BlockSpec((TT, NH * HD), lambda t: (t, 0)),
            pl.BlockSpec((NH * HD, HID), lambda t: (0, 0)),
        ],
        out_specs=pl.BlockSpec((TT, HID), lambda t: (t, 0)),
        out_shape=jax.ShapeDtypeStruct((T, HID), F32),
        compiler_params=pltpu.CompilerParams(
            dimension_semantics=("parallel",)),
    )(attn.transpose(0, 2, 1, 3).reshape(T, NH * HD), wo.astype(BF))

    gate_loss = klp[:, 0, 0].sum() * (1.0 / (NKV * SEQ * KB * BSZ))
    return attn_output, gate_loss


# speedup vs baseline: 1.3413x; 1.0080x over previous
import math

import jax
import jax.numpy as jnp
from jax.experimental import pallas as pl
from jax.experimental.pallas import tpu as pltpu

BSZ = 2
SEQ = 2048
T = BSZ * SEQ
HID = 2048
NH = 16
NKV = 8
G = NH // NKV
HD = 128
BLK = 64
KB = SEQ // BLK
GH = 128
EPS = 1e-6

TT = 512
QB = 512
CK = 512
NQB = SEQ // QB
BPT = TT // BLK
BF = jnp.bfloat16
F32 = jnp.float32


def _rot(x):
    h = HD // 2
    return jnp.concatenate([-x[..., h:], x[..., :h]], axis=-1)


def _proj_kernel(h_ref, cos_ref, sin_ref, cg_ref, sg_ref, bc_ref, bs_ref,
                 wq_ref, wk_ref, wv_ref, qw_ref, kw_ref, gwq_ref, gwk_ref,
                 qr_ref, kr_ref, v_ref, qg_ref, kg_ref):
    h = h_ref[...]
    cos = cos_ref[...]
    sin = sin_ref[...]

    q = jnp.dot(h, wq_ref[...], preferred_element_type=F32)
    q3 = q.reshape(TT, NH, HD)
    var = jnp.mean(q3 * q3, axis=-1, keepdims=True)
    qn = q3 * jax.lax.rsqrt(var + EPS) * qw_ref[0][None, None, :]
    qr = qn * cos[:, None, :] + _rot(qn) * sin[:, None, :]
    qr_ref[...] = qr.reshape(TT, NH * HD).astype(BF)

    qg = qn.reshape(TT, NKV, G, HD).mean(axis=2)
    qg = qg * cg_ref[...][:, None, :] + _rot(qg) * sg_ref[...][:, None, :]
    qgp = jnp.dot(qg.reshape(TT * NKV, HD).astype(BF), gwq_ref[...],
                  preferred_element_type=F32)
    qg_ref[...] = qgp.reshape(TT, NKV * GH).astype(BF)

    k = jnp.dot(h, wk_ref[...], preferred_element_type=F32)
    k3 = k.reshape(TT, NKV, HD)
    kvar = jnp.mean(k3 * k3, axis=-1, keepdims=True)
    kn = k3 * jax.lax.rsqrt(kvar + EPS) * kw_ref[0][None, None, :]
    kr = kn * cos[:, None, :] + _rot(kn) * sin[:, None, :]
    kr_ref[...] = kr.reshape(TT, NKV * HD).astype(BF)

    kb4 = kn.reshape(BPT, BLK, NKV, HD)
    kmax = kb4.max(axis=1)
    kavg = kb4.mean(axis=1)
    bc = bc_ref[:, 0, :]
    bs = bs_ref[:, 0, :]
    kmax = kmax * bc[:, None, :] + _rot(kmax) * bs[:, None, :]
    kavg = kavg * bc[:, None, :] + _rot(kavg) * bs[:, None, :]
    kcat = jnp.concatenate([kmax, kavg], axis=-1).reshape(BPT * NKV, 2 * HD)
    kgp = jnp.dot(kcat.astype(BF), gwk_ref[...], preferred_element_type=F32)
    kg_ref[:, 0, :] = kgp.reshape(BPT, NKV * GH).astype(BF)

    v_ref[...] = jnp.dot(h, wv_ref[...],
                         preferred_element_type=F32).astype(BF)


def _flash_kernel(q_ref, k_ref, v_ref, qg_ref, kg_ref, o_ref, kl_ref):
    qb = pl.program_id(1)
    scale = 1.0 / math.sqrt(HD)
    rows = G * QB

    qblk = q_ref[0, 0, :, :]
    qs = jnp.concatenate([qblk[:, :HD], qblk[:, HD:]], axis=0)

    ind = (jax.lax.broadcasted_iota(jnp.int32, (CK, CK // BLK), 1)
           == jax.lax.broadcasted_iota(jnp.int32, (CK, CK // BLK), 0)
           // BLK).astype(BF)

    def chunk(j, m, l, acc, psum, masked):
        kc = k_ref[0, 0, pl.ds(j * CK, CK), :]
        vc = v_ref[0, 0, pl.ds(j * CK, CK), :]
        s = jax.lax.dot_general(qs, kc, (((1,), (1,)), ((), ())),
                                preferred_element_type=F32) * scale
        if masked:
            rq = jax.lax.broadcasted_iota(jnp.int32, (rows, CK), 0) % QB
            ck = jax.lax.broadcasted_iota(jnp.int32, (rows, CK), 1)
            s = jnp.where(ck <= rq, s, -1e30)
        m_new = jnp.maximum(m, s.max(axis=-1, keepdims=True))
        corr = jnp.exp(m - m_new)
        p = jnp.exp(s - m_new)
        pb = p.astype(BF)
        l = l * corr + p.sum(axis=-1, keepdims=True)
        acc = acc * corr + jnp.dot(pb, vc, preferred_element_type=F32)
        ps = jnp.dot(pb, ind, preferred_element_type=F32)
        colid = jax.lax.broadcasted_iota(jnp.int32, (rows, KB), 1)
        upd = jnp.zeros((rows, KB), dtype=F32)
        for c in range(CK // BLK):
            upd = upd + jnp.where(colid == j * (CK // BLK) + c,
                                  ps[:, c][:, None], 0.0)
        psum = psum * corr + upd
        return m_new, l, acc, psum

    m0 = jnp.full((rows, 1), -1e30, dtype=F32)
    l0 = jnp.zeros((rows, 1), dtype=F32)
    a0 = jnp.zeros((rows, HD), dtype=F32)
    p0 = jnp.zeros((rows, KB), dtype=F32)

    def body(j, carry):
        return chunk(j, *carry, masked=False)

    m, l, acc, psum = jax.lax.fori_loop(0, qb, body, (m0, l0, a0, p0))
    m, l, acc, psum = chunk(qb, m, l, acc, psum, masked=True)

    attn = acc / l
    o_ref[0, 0, :, :] = jnp.concatenate(
        [attn[:QB], attn[QB:]], axis=1).astype(BF)

    pn = psum / l
    m1d = jnp.maximum(pn[:QB], pn[QB:])
    gt = m1d / (m1d.sum(axis=-1, keepdims=True) + 1e-9)

    qg = qg_ref[0, 0, :, :]
    kg = kg_ref[0, 0, :, :]
    logits = jax.lax.dot_general(qg, kg, (((1,), (1,)), ((), ())),
                                 preferred_element_type=F32)
    logits = logits * (1.0 / math.sqrt(GH))
    rowq = jax.lax.broadcasted_iota(jnp.int32, (QB, KB), 0) + qb * QB
    colb = jax.lax.broadcasted_iota(jnp.int32, (QB, KB), 1) * BLK
    x = jnp.where(colb <= rowq, logits, -1e30)
    xm = x.max(axis=-1, keepdims=True)
    pm = x - xm - jnp.log(jnp.exp(x - xm).sum(axis=-1, keepdims=True))

    gt_safe = jnp.where(gt > 0, gt, 1.0)
    kl = jnp.where(gt > 0, gt * (jnp.log(gt_safe) - pm), 0.0)
    kl_ref[0, 0, :] = jnp.full((GH,), kl.sum(), dtype=F32)


def _out_kernel(x_ref, wo_ref, o_ref):
    o_ref[...] = jnp.dot(x_ref[...], wo_ref[...],
                         preferred_element_type=F32)


def kernel(hidden_states, cos, sin, cos_gate, sin_gate, block_cos, block_sin,
           block_attention_mask, cu_seqlens, wq, wk, wv, wo,
           q_norm_w, k_norm_w, gate_wq, gate_wk):
    nt = T // TT
    ntb = SEQ // TT

    qr, kr, v, qg, kg = pl.pallas_call(
        _proj_kernel,
        grid=(nt,),
        in_specs=[
            pl.BlockSpec((TT, HID), lambda t: (t, 0)),
            pl.BlockSpec((TT, HD), lambda t: (t, 0)),
            pl.BlockSpec((TT, HD), lambda t: (t, 0)),
            pl.BlockSpec((TT, HD), lambda t: (t, 0)),
            pl.BlockSpec((TT, HD), lambda t: (t, 0)),
            pl.BlockSpec((BPT, 1, HD), lambda t: (t % (SEQ // TT), 0, 0)),
            pl.BlockSpec((BPT, 1, HD), lambda t: (t % (SEQ // TT), 0, 0)),
            pl.BlockSpec((HID, NH * HD), lambda t: (0, 0)),
            pl.BlockSpec((HID, NKV * HD), lambda t: (0, 0)),
            pl.BlockSpec((HID, NKV * HD), lambda t: (0, 0)),
            pl.BlockSpec((1, HD), lambda t: (0, 0)),
            pl.BlockSpec((1, HD), lambda t: (0, 0)),
            pl.BlockSpec((HD, GH), lambda t: (0, 0)),
            pl.BlockSpec((2 * HD, GH), lambda t: (0, 0)),
        ],
        out_specs=[
            pl.BlockSpec((TT, NH * HD), lambda t: (t, 0)),
            pl.BlockSpec((TT, NKV * HD), lambda t: (t, 0)),
            pl.BlockSpec((TT, NKV * HD), lambda t: (t, 0)),
            pl.BlockSpec((TT, NKV * GH), lambda t: (t, 0)),
            pl.BlockSpec((BPT, 1, NKV * GH), lambda t: (t, 0, 0)),
        ],
        out_shape=[
            jax.ShapeDtypeStruct((T, NH * HD), BF),
            jax.ShapeDtypeStruct((T, NKV * HD), BF),
            jax.ShapeDtypeStruct((T, NKV * HD), BF),
            jax.ShapeDtypeStruct((T, NKV * GH), BF),
            jax.ShapeDtypeStruct((BSZ * KB, 1, NKV * GH), BF),
        ],
        compiler_params=pltpu.CompilerParams(
            dimension_semantics=("parallel",)),
    )(hidden_states.astype(BF), cos, sin, cos_gate, sin_gate,
      block_cos.reshape(KB, 1, HD), block_sin.reshape(KB, 1, HD),
      wq.astype(BF), wk.astype(BF), wv.astype(BF),
      q_norm_w.reshape(1, HD), k_norm_w.reshape(1, HD),
      gate_wq.astype(BF), gate_wk.astype(BF))

    qr4 = qr.reshape(BSZ, SEQ, NKV, G * HD).transpose(0, 2, 1, 3)
    kr4 = kr.reshape(BSZ, SEQ, NKV, HD).transpose(0, 2, 1, 3)
    v4 = v.reshape(BSZ, SEQ, NKV, HD).transpose(0, 2, 1, 3)
    qg4 = qg.reshape(BSZ, SEQ, NKV, GH).transpose(0, 2, 1, 3)
    kg4 = kg.reshape(BSZ, KB, NKV, GH).transpose(0, 2, 1, 3)

    attn, klp = pl.pallas_call(
        _flash_kernel,
        grid=(BSZ * NKV, NQB),
        in_specs=[
            pl.BlockSpec((1, 1, QB, G * HD),
                         lambda bn, qb: (bn // NKV, bn % NKV, qb, 0)),
            pl.BlockSpec((1, 1, SEQ, HD),
                         lambda bn, qb: (bn // NKV, bn % NKV, 0, 0)),
            pl.BlockSpec((1, 1, SEQ, HD),
                         lambda bn, qb: (bn // NKV, bn % NKV, 0, 0)),
            pl.BlockSpec((1, 1, QB, GH),
                         lambda bn, qb: (bn // NKV, bn % NKV, qb, 0)),
            pl.BlockSpec((1, 1, KB, GH),
                         lambda bn, qb: (bn // NKV, bn % NKV, 0, 0)),
        ],
        out_specs=[
            pl.BlockSpec((1, 1, QB, G * HD),
                         lambda bn, qb: (bn // NKV, bn % NKV, qb, 0)),
            pl.BlockSpec((1, 1, GH), lambda bn, qb: (bn * NQB + qb, 0, 0)),
        ],
        out_shape=[
            jax.ShapeDtypeStruct((BSZ, NKV, SEQ, G * HD), BF),
            jax.ShapeDtypeStruct((BSZ * NKV * NQB, 1, GH), F32),
        ],
        compiler_params=pltpu.CompilerParams(
            dimension_semantics=("parallel", "arbitrary")),
    )(qr4, kr4, v4, qg4, kg4)

    attn_output = pl.pallas_call(
        _out_kernel,
        grid=(nt,),
        in_specs=[
            pl.BlockSpec((TT, NH * HD), lambda t: (t, 0)),
            pl.BlockSpec((NH * HD, HID), lambda t: (0, 0)),
        ],
        out_specs=pl.BlockSpec((TT, HID), lambda t: (t, 0)),
        out_shape=jax.ShapeDtypeStruct((T, HID), F32),
        compiler_params=pltpu.CompilerParams(
            dimension_semantics=("parallel",)),
    )(attn.transpose(0, 2, 1, 3).reshape(T, NH * HD), wo.astype(BF))

    gate_loss = klp[:, 0, 0].sum() * (1.0 / (NKV * SEQ * KB * BSZ))
    return attn_output, gate_loss
